# baseline (speedup 1.0000x reference)
import numpy as np
import jax
import jax.numpy as jnp
from math import factorial
from functools import partial
from jax.experimental import pallas as pl
from jax.experimental.pallas import tpu as pltpu

_np_asarray_orig = jnp.asarray


def _asarray_keep_complex_host(a, *args, **kwargs):
    if isinstance(a, np.ndarray) and np.iscomplexobj(a):
        return a
    return _np_asarray_orig(a, *args, **kwargs)


jnp.asarray = _asarray_keep_complex_host

_LS = list(range(9))
_DIM = 81
_NE = 1024
_BT = 256


def _build_tables():
    import ml_dtypes
    WC = np.zeros((81, 2 * _NE), dtype=np.float64)
    TSA = np.zeros((18, 2 * _NE), dtype=np.float32)
    TSG = np.zeros((18, 2 * _NE), dtype=np.float32)
    EA = np.zeros(81, dtype=np.float32)
    EB = np.zeros(81, dtype=np.float32)
    rows = []
    off = 0
    base = 0
    for l in _LS:
        n = 2 * l + 1
        for j in range(n):
            EA[l * l + j] = 2 * l - j
            EB[l * l + j] = j
        dcoef = np.zeros((n, n, n))
        for mp in range(-l, l + 1):
            for m in range(-l, l + 1):
                kmin = max(0, m - mp)
                kmax = min(l + m, l - mp)
                for k in range(kmin, kmax + 1):
                    num = np.sqrt(float(factorial(l + mp) * factorial(l - mp)
                                        * factorial(l + m) * factorial(l - m)))
                    den = float(factorial(l + m - k) * factorial(k)
                                * factorial(l - mp - k) * factorial(mp - m + k))
                    co = ((-1.0) ** (mp - m + k)) * num / den
                    dcoef[l + mp, l + m, mp - m + 2 * k] += co
        for r, p in enumerate(range(-l, l + 1)):
            rows.append((l, off, r, base + r * n))
            for cidx, q in enumerate(range(-l, l + 1)):
                mu, nu = abs(p), abs(q)
                pref = 0.5 * (2.0 ** -0.5 if mu == 0 else 1.0) \
                           * (2.0 ** -0.5 if nu == 0 else 1.0)
                sPP = (-1.0) ** (mu + nu)
                sPM = (-1.0) ** mu
                sMP = (-1.0) ** nu
                dPP = dcoef[l + mu, l + nu]; dPM = dcoef[l + mu, l - nu]
                dMP = dcoef[l - mu, l + nu]; dMM = dcoef[l - mu, l - nu]
                Xp = pref * (sPP * dPP + sPM * dPM + sMP * dMP + dMM)
                Yp = pref * (sPP * dPP - sPM * dPM - sMP * dMP + dMM)
                e = base + r * n + cidx
                WC[l * l:l * l + n, e] = Xp
                WC[l * l:l * l + n, _NE + e] = Yp
                if p >= 0:
                    TSA[mu, e] = 1.0
                    TSA[9 + mu, _NE + e] = 1.0
                else:
                    TSA[9 + mu, e] = -1.0
                    TSA[mu, _NE + e] = 1.0
                if q >= 0:
                    TSG[nu, e] = 1.0
                    TSG[9 + nu, _NE + e] = -1.0
                else:
                    TSG[9 + nu, e] = 1.0
                    TSG[nu, _NE + e] = 1.0
        base += n * n
        off += n
    bf16 = ml_dtypes.bfloat16
    wc32 = WC.astype(np.float32)
    wh = wc32.astype(bf16)
    wl = (wc32 - wh.astype(np.float32)).astype(bf16)
    W3 = np.concatenate([wh, wl, wh], axis=0)
    EXPM = np.concatenate(
        [EA[None], EB[None],
         np.pad(np.arange(9, dtype=np.float32), (0, 72))[None]], axis=0)
    TS2 = np.stack([TSA, TSG]).astype(bf16)
    return W3, TS2, EXPM.astype(np.float32), rows


_W3, _TS2, _EXPM, _ROWS = _build_tables()


def _body(a_ref, b_ref, g_ref, w3_ref, ts_ref, exp_ref, out_ref):
    a = a_ref[:]
    b = b_ref[:]
    g = g_ref[:]
    c = jnp.cos(0.5 * b)
    s = jnp.sin(0.5 * b)
    lc = jnp.log(jnp.maximum(c, 1e-30))
    ls = jnp.log(jnp.maximum(s, 1e-30))
    ea = exp_ref[0:1, :]
    eb = exp_ref[1:2, :]
    mono = jnp.exp(ea * lc + eb * ls)
    mh = mono.astype(jnp.bfloat16)
    mlo = (mono - mh.astype(jnp.float32)).astype(jnp.bfloat16)
    mono3 = jnp.concatenate([mh, mh, mlo], axis=1)
    XY = jnp.dot(mono3, w3_ref[:], preferred_element_type=jnp.float32)

    mus = exp_ref[2:3, 0:9]
    am_ = a * mus
    gm_ = g * mus
    CAS = jnp.concatenate([jnp.cos(am_), jnp.sin(am_)], axis=1)
    CGS = jnp.concatenate([jnp.cos(gm_), jnp.sin(gm_)], axis=1)
    cash = CAS.astype(jnp.bfloat16)
    casl = (CAS - cash.astype(jnp.float32)).astype(jnp.bfloat16)
    cgsh = CGS.astype(jnp.bfloat16)
    cgsl = (CGS - cgsh.astype(jnp.float32)).astype(jnp.bfloat16)
    tsa = ts_ref[0]
    tsg = ts_ref[1]
    Asel = (jnp.dot(jnp.concatenate([cash, casl], axis=1),
                    jnp.concatenate([tsa, tsa], axis=0),
                    preferred_element_type=jnp.float32))
    Gsel = (jnp.dot(jnp.concatenate([cgsh, cgsl], axis=1),
                    jnp.concatenate([tsg, tsg], axis=0),
                    preferred_element_type=jnp.float32))
    OC2 = Asel * Gsel * XY
    OC = OC2[:, 0:_NE] + OC2[:, _NE:2 * _NE]

    out_ref[:] = jnp.zeros((out_ref.shape[0], _DIM, _DIM), jnp.float32)
    for l, off, r, lane in _ROWS:
        n = 2 * l + 1
        out_ref[:, off + r, pl.ds(off, n)] = OC[:, lane:lane + n]


@jax.jit
def kernel(alpha, beta, gamma):
    B = alpha.shape[0]
    nbt = B // _BT
    a2 = alpha.reshape(B, 1)
    b2 = beta.reshape(B, 1)
    g2 = gamma.reshape(B, 1)
    angle_spec = pl.BlockSpec((_BT, 1), lambda i: (i, 0))
    constw = pl.BlockSpec((243, 2 * _NE), lambda i: (0, 0))
    constt = pl.BlockSpec((2, 18, 2 * _NE), lambda i: (0, 0, 0))
    conste = pl.BlockSpec((3, 81), lambda i: (0, 0))
    return pl.pallas_call(
        _body,
        grid=(nbt,),
        in_specs=[angle_spec, angle_spec, angle_spec, constw, constt, conste],
        out_specs=pl.BlockSpec((_BT, _DIM, _DIM), lambda i: (i, 0, 0)),
        out_shape=jax.ShapeDtypeStruct((B, _DIM, _DIM), jnp.float32),
    )(a2, b2, g2, _W3, _TS2, _EXPM)


# speedup vs baseline: 1.2054x; 1.2054x over previous
import numpy as np
import jax
import jax.numpy as jnp
from math import factorial
from functools import partial
from jax.experimental import pallas as pl
from jax.experimental.pallas import tpu as pltpu

_np_asarray_orig = jnp.asarray


def _asarray_keep_complex_host(a, *args, **kwargs):
    if isinstance(a, np.ndarray) and np.iscomplexobj(a):
        return a
    return _np_asarray_orig(a, *args, **kwargs)


jnp.asarray = _asarray_keep_complex_host

_LS = list(range(9))
_DIM = 81
_BATCH = 4096
_BT = 64


def _build_tables():
    WX = np.zeros((81, _DIM * _DIM), dtype=np.float64)
    WY = np.zeros((81, _DIM * _DIM), dtype=np.float64)
    SAp = np.zeros((18, _DIM)); SAm = np.zeros((18, _DIM))
    SGp = np.zeros((18, _DIM)); SGm = np.zeros((18, _DIM))
    EA = np.zeros(81); EB = np.zeros(81)
    off = 0
    for l in _LS:
        n = 2 * l + 1
        for j in range(n):
            EA[l * l + j] = 2 * l - j
            EB[l * l + j] = j
        dcoef = np.zeros((n, n, n))
        for mp in range(-l, l + 1):
            for m in range(-l, l + 1):
                kmin = max(0, m - mp)
                kmax = min(l + m, l - mp)
                for k in range(kmin, kmax + 1):
                    num = np.sqrt(float(factorial(l + mp) * factorial(l - mp)
                                        * factorial(l + m) * factorial(l - m)))
                    den = float(factorial(l + m - k) * factorial(k)
                                * factorial(l - mp - k) * factorial(mp - m + k))
                    co = ((-1.0) ** (mp - m + k)) * num / den
                    dcoef[l + mp, l + m, mp - m + 2 * k] += co
        for p in range(-l, l + 1):
            i = off + l + p
            mu = abs(p)
            SAp[mu if p >= 0 else 9 + mu, i] = 1.0 if p >= 0 else -1.0
            SAm[9 + mu if p >= 0 else mu, i] = 1.0
        for q in range(-l, l + 1):
            jj = off + l + q
            nu = abs(q)
            SGp[nu if q >= 0 else 9 + nu, jj] = 1.0
            SGm[9 + nu if q >= 0 else nu, jj] = -1.0 if q >= 0 else 1.0
        for p in range(-l, l + 1):
            for q in range(-l, l + 1):
                mu, nu = abs(p), abs(q)
                pref = 0.5 * (2.0 ** -0.5 if mu == 0 else 1.0) \
                           * (2.0 ** -0.5 if nu == 0 else 1.0)
                sPP = (-1.0) ** (mu + nu); sPM = (-1.0) ** mu; sMP = (-1.0) ** nu
                dPP = dcoef[l + mu, l + nu]; dPM = dcoef[l + mu, l - nu]
                dMP = dcoef[l - mu, l + nu]; dMM = dcoef[l - mu, l - nu]
                Xp = pref * (sPP * dPP + sPM * dPM + sMP * dMP + dMM)
                Yp = pref * (sPP * dPP - sPM * dPM - sMP * dMP + dMM)
                col = 81 * (off + l + p) + (off + l + q)
                WX[l * l:l * l + n, col] = Xp
                WY[l * l:l * l + n, col] = Yp
        off += n
    f32 = np.float32
    return (WX.astype(f32), WY.astype(f32), SAp.astype(f32), SAm.astype(f32),
            SGp.astype(f32), SGm.astype(f32), EA.astype(f32), EB.astype(f32))


_WX, _WY, _SAp, _SAm, _SGp, _SGm, _EA, _EB = _build_tables()
_MUS = np.arange(9, dtype=np.float32)
_SEL = np.stack([_SAp, _SAm, _SGp, _SGm])


def _split_bf16(w):
    import ml_dtypes
    hi = w.astype(ml_dtypes.bfloat16)
    lo = (w - hi.astype(np.float32)).astype(ml_dtypes.bfloat16)
    return hi, lo


def _build_w3():
    import ml_dtypes
    ncol = 6656
    wxh, wxl = _split_bf16(_WX)
    wyh, wyl = _split_bf16(_WY)
    w3 = np.zeros((243, 2 * ncol), dtype=ml_dtypes.bfloat16)
    w3[0:81, 0:6561] = wxh
    w3[81:162, 0:6561] = wxl
    w3[162:243, 0:6561] = wxh
    w3[0:81, ncol:ncol + 6561] = wyh
    w3[81:162, ncol:ncol + 6561] = wyl
    w3[162:243, ncol:ncol + 6561] = wyh
    return w3


_W3 = _build_w3()
_EXPM = np.concatenate(
    [_EA[None], _EB[None], np.pad(_MUS, (0, 72))[None]], axis=0
).astype(np.float32)


def _body(a_ref, b_ref, g_ref, w3_ref, sel_ref, exp_ref, out_ref):
    a = a_ref[:]
    b = b_ref[:]
    g = g_ref[:]
    c = jnp.cos(0.5 * b)
    s = jnp.sin(0.5 * b)
    lc = jnp.log(jnp.maximum(c, 1e-30))
    ls = jnp.log(jnp.maximum(s, 1e-30))
    ea = exp_ref[0:1, :]
    eb = exp_ref[1:2, :]
    mono = jnp.exp(ea * lc + eb * ls)
    mus = exp_ref[2:3, 0:9]
    am_ = a * mus
    gm_ = g * mus
    CAS = jnp.concatenate([jnp.cos(am_), jnp.sin(am_)], axis=1)
    CGS = jnp.concatenate([jnp.cos(gm_), jnp.sin(gm_)], axis=1)
    sel = sel_ref[:]
    dot = partial(jnp.dot, preferred_element_type=jnp.float32,
                  precision=jax.lax.Precision.HIGHEST)
    Ap = dot(CAS, sel[0])
    Am = dot(CAS, sel[1])
    Gp = dot(CGS, sel[2])
    Gm = dot(CGS, sel[3])
    mh = mono.astype(jnp.bfloat16)
    mlo = (mono - mh.astype(jnp.float32)).astype(jnp.bfloat16)
    mono3 = jnp.concatenate([mh, mh, mlo], axis=1)
    XY = jnp.dot(mono3, w3_ref[:], preferred_element_type=jnp.float32)
    X = XY[:, 0:6561].reshape(_BT, _DIM, _DIM)
    Y = XY[:, 6656:6656 + 6561].reshape(_BT, _DIM, _DIM)
    out_ref[:] = (Ap[:, :, None] * X * Gp[:, None, :]
                  + Am[:, :, None] * Y * Gm[:, None, :])


@jax.jit
def kernel(alpha, beta, gamma):
    B = alpha.shape[0]
    nbt = B // _BT
    a2 = alpha.reshape(B, 1)
    b2 = beta.reshape(B, 1)
    g2 = gamma.reshape(B, 1)
    angle_spec = pl.BlockSpec((_BT, 1), lambda i: (i, 0))
    constw = pl.BlockSpec((243, 13312), lambda i: (0, 0))
    const3 = pl.BlockSpec((4, 18, _DIM), lambda i: (0, 0, 0))
    conste = pl.BlockSpec((3, 81), lambda i: (0, 0))
    return pl.pallas_call(
        _body,
        grid=(nbt,),
        in_specs=[angle_spec, angle_spec, angle_spec, constw, const3, conste],
        out_specs=pl.BlockSpec((_BT, _DIM, _DIM), lambda i: (i, 0, 0)),
        out_shape=jax.ShapeDtypeStruct((B, _DIM, _DIM), jnp.float32),
    )(a2, b2, g2, _W3, _SEL, _EXPM)


# speedup vs baseline: 1.7654x; 1.4646x over previous
import numpy as np
import jax
import jax.numpy as jnp
from math import factorial
from functools import partial
from jax.experimental import pallas as pl
from jax.experimental.pallas import tpu as pltpu

_np_asarray_orig = jnp.asarray


def _asarray_keep_complex_host(a, *args, **kwargs):
    if isinstance(a, np.ndarray) and np.iscomplexobj(a):
        return a
    return _np_asarray_orig(a, *args, **kwargs)


jnp.asarray = _asarray_keep_complex_host

_LS = list(range(9))
_DIM = 81
_NE = 1024
_BT = 128


def _build_tables():
    import ml_dtypes
    WC = np.zeros((81, 2 * _NE), dtype=np.float64)
    TSA = np.zeros((18, 2 * _NE), dtype=np.float32)
    TSG = np.zeros((18, 2 * _NE), dtype=np.float32)
    EA = np.zeros(81, dtype=np.float32)
    EB = np.zeros(81, dtype=np.float32)
    blocks = []
    off = 0
    base = 0
    for l in _LS:
        n = 2 * l + 1
        blocks.append((l, off, base))
        for j in range(n):
            EA[l * l + j] = 2 * l - j
            EB[l * l + j] = j
        dcoef = np.zeros((n, n, n))
        for mp in range(-l, l + 1):
            for m in range(-l, l + 1):
                kmin = max(0, m - mp)
                kmax = min(l + m, l - mp)
                for k in range(kmin, kmax + 1):
                    num = np.sqrt(float(factorial(l + mp) * factorial(l - mp)
                                        * factorial(l + m) * factorial(l - m)))
                    den = float(factorial(l + m - k) * factorial(k)
                                * factorial(l - mp - k) * factorial(mp - m + k))
                    co = ((-1.0) ** (mp - m + k)) * num / den
                    dcoef[l + mp, l + m, mp - m + 2 * k] += co
        for r, p in enumerate(range(-l, l + 1)):
            for cidx, q in enumerate(range(-l, l + 1)):
                mu, nu = abs(p), abs(q)
                pref = 0.5 * (2.0 ** -0.5 if mu == 0 else 1.0) \
                           * (2.0 ** -0.5 if nu == 0 else 1.0)
                sPP = (-1.0) ** (mu + nu)
                sPM = (-1.0) ** mu
                sMP = (-1.0) ** nu
                dPP = dcoef[l + mu, l + nu]; dPM = dcoef[l + mu, l - nu]
                dMP = dcoef[l - mu, l + nu]; dMM = dcoef[l - mu, l - nu]
                Xp = pref * (sPP * dPP + sPM * dPM + sMP * dMP + dMM)
                Yp = pref * (sPP * dPP - sPM * dPM - sMP * dMP + dMM)
                e = base + r * n + cidx
                WC[l * l:l * l + n, e] = Xp
                WC[l * l:l * l + n, _NE + e] = Yp
                if p >= 0:
                    TSA[mu, e] = 1.0
                    TSA[9 + mu, _NE + e] = 1.0
                else:
                    TSA[9 + mu, e] = -1.0
                    TSA[mu, _NE + e] = 1.0
                if q >= 0:
                    TSG[nu, e] = 1.0
                    TSG[9 + nu, _NE + e] = -1.0
                else:
                    TSG[9 + nu, e] = 1.0
                    TSG[nu, _NE + e] = 1.0
        base += n * n
        off += n
    bf16 = ml_dtypes.bfloat16
    wc32 = WC.astype(np.float32)
    wh = wc32.astype(bf16)
    wl = (wc32 - wh.astype(np.float32)).astype(bf16)
    W3 = np.concatenate([wh, wl, wh], axis=0)
    EXPM = np.concatenate(
        [EA[None], EB[None],
         np.pad(np.arange(9, dtype=np.float32), (0, 72))[None]], axis=0)
    TS2 = np.stack([TSA, TSG]).astype(bf16)
    return W3, TS2, EXPM.astype(np.float32), blocks


_W3, _TS2, _EXPM, _BLOCKS = _build_tables()


def _body(a_ref, b_ref, g_ref, w3_ref, ts_ref, exp_ref, out_ref):
    a = a_ref[:]
    b = b_ref[:]
    g = g_ref[:]
    c = jnp.cos(0.5 * b)
    s = jnp.sin(0.5 * b)
    lc = jnp.log(jnp.maximum(c, 1e-30))
    ls = jnp.log(jnp.maximum(s, 1e-30))
    ea = exp_ref[0:1, :]
    eb = exp_ref[1:2, :]
    mono = jnp.exp(ea * lc + eb * ls)
    mh = mono.astype(jnp.bfloat16)
    mlo = (mono - mh.astype(jnp.float32)).astype(jnp.bfloat16)
    mono3 = jnp.concatenate([mh, mh, mlo], axis=1)
    XY = jnp.dot(mono3, w3_ref[:], preferred_element_type=jnp.float32)

    mus = exp_ref[2:3, 0:9]
    am_ = a * mus
    gm_ = g * mus
    CAS = jnp.concatenate([jnp.cos(am_), jnp.sin(am_)], axis=1)
    CGS = jnp.concatenate([jnp.cos(gm_), jnp.sin(gm_)], axis=1)
    cash = CAS.astype(jnp.bfloat16)
    casl = (CAS - cash.astype(jnp.float32)).astype(jnp.bfloat16)
    cgsh = CGS.astype(jnp.bfloat16)
    cgsl = (CGS - cgsh.astype(jnp.float32)).astype(jnp.bfloat16)
    tsa = ts_ref[0]
    tsg = ts_ref[1]
    Asel = jnp.dot(jnp.concatenate([cash, casl], axis=1),
                   jnp.concatenate([tsa, tsa], axis=0),
                   preferred_element_type=jnp.float32)
    Gsel = jnp.dot(jnp.concatenate([cgsh, cgsl], axis=1),
                   jnp.concatenate([tsg, tsg], axis=0),
                   preferred_element_type=jnp.float32)
    OC2 = Asel * Gsel * XY
    OC = OC2[:, 0:_NE] + OC2[:, _NE:2 * _NE]

    out_ref[:] = jnp.zeros((out_ref.shape[0], _DIM, _DIM), jnp.float32)
    for l, off, base in _BLOCKS:
        n = 2 * l + 1
        blk = OC[:, base:base + n * n].reshape(out_ref.shape[0], n, n)
        out_ref[:, pl.ds(off, n), pl.ds(off, n)] = blk


@jax.jit
def kernel(alpha, beta, gamma):
    B = alpha.shape[0]
    nbt = B // _BT
    a2 = alpha.reshape(B, 1)
    b2 = beta.reshape(B, 1)
    g2 = gamma.reshape(B, 1)
    angle_spec = pl.BlockSpec((_BT, 1), lambda i: (i, 0))
    constw = pl.BlockSpec((243, 2 * _NE), lambda i: (0, 0))
    constt = pl.BlockSpec((2, 18, 2 * _NE), lambda i: (0, 0, 0))
    conste = pl.BlockSpec((3, 81), lambda i: (0, 0))
    return pl.pallas_call(
        _body,
        grid=(nbt,),
        in_specs=[angle_spec, angle_spec, angle_spec, constw, constt, conste],
        out_specs=pl.BlockSpec((_BT, _DIM, _DIM), lambda i: (i, 0, 0)),
        out_shape=jax.ShapeDtypeStruct((B, _DIM, _DIM), jnp.float32),
    )(a2, b2, g2, _W3, _TS2, _EXPM)
